# Initial kernel scaffold; baseline (speedup 1.0000x reference)
import jax
import jax.numpy as jnp
from jax.experimental import pallas as pl


def kernel(X, E1, E2, theta1, theta2):
    raise NotImplementedError("write your pallas kernel here")


# speedup vs baseline: 17.9316x; 17.9316x over previous
import functools

import jax
import jax.numpy as jnp
from jax.experimental import pallas as pl
from jax.experimental.pallas import tpu as pltpu

N = 2048
D = 64
K = 16
ALPHA = 0.2
ROWS = 256


def _prep_kernel(x_ref, e1_ref, e2_ref, t1_ref, t2_ref,
                 xn_ref, m1_ref, m2_ref):
    x = x_ref[0]
    acc = x[0]
    for t in range(1, x.shape[0]):
        acc = acc + x[t]
    xm = acc / x.shape[0]
    norm = jnp.sqrt(jnp.sum(xm * xm, axis=-1, keepdims=True))
    xn_ref[0] = xm / jnp.maximum(norm, 1e-8)
    m1_ref[...] = jnp.tanh(ALPHA * jnp.dot(e1_ref[...], t1_ref[...],
                                           preferred_element_type=jnp.float32))
    m2_ref[...] = jnp.tanh(ALPHA * jnp.dot(e2_ref[...], t2_ref[...],
                                           preferred_element_type=jnp.float32))


def _main_kernel(xnr_ref, xnf_ref, m1r_ref, m1f_ref, m2r_ref, m2f_ref,
                 out_ref):
    xr = xnr_ref[0]
    xf = xnf_ref[0]
    sim = jax.lax.dot_general(xr, xf, (((1,), (1,)), ((), ())),
                              preferred_element_type=jnp.float32)
    v = sim
    for _ in range(K - 1):
        m = jnp.max(v, axis=-1, keepdims=True)
        v = jnp.where(v >= m, -2.0, v)
    thresh = jnp.max(v, axis=-1, keepdims=True)
    a = jax.lax.dot_general(m1r_ref[...], m2f_ref[...], (((1,), (1,)), ((), ())),
                            preferred_element_type=jnp.float32)
    bm = jax.lax.dot_general(m2r_ref[...], m1f_ref[...], (((1,), (1,)), ((), ())),
                             preferred_element_type=jnp.float32)
    dyn = jax.nn.relu(jnp.tanh(ALPHA * (a - bm)))
    out_ref[0] = jnp.where(sim >= thresh, dyn * sim, 0.0)


@jax.jit
def kernel(X, E1, E2, theta1, theta2):
    B, T = X.shape[0], X.shape[1]
    xn, m1, m2 = pl.pallas_call(
        _prep_kernel,
        grid=(B,),
        in_specs=[
            pl.BlockSpec((1, T, N, D), lambda b: (b, 0, 0, 0)),
            pl.BlockSpec((N, D), lambda b: (0, 0)),
            pl.BlockSpec((N, D), lambda b: (0, 0)),
            pl.BlockSpec((D, D), lambda b: (0, 0)),
            pl.BlockSpec((D, D), lambda b: (0, 0)),
        ],
        out_specs=[
            pl.BlockSpec((1, N, D), lambda b: (b, 0, 0)),
            pl.BlockSpec((N, D), lambda b: (0, 0)),
            pl.BlockSpec((N, D), lambda b: (0, 0)),
        ],
        out_shape=[
            jax.ShapeDtypeStruct((B, N, D), jnp.float32),
            jax.ShapeDtypeStruct((N, D), jnp.float32),
            jax.ShapeDtypeStruct((N, D), jnp.float32),
        ],
    )(X, E1, E2, theta1, theta2)

    out = pl.pallas_call(
        _main_kernel,
        grid=(B, N // ROWS),
        in_specs=[
            pl.BlockSpec((1, ROWS, D), lambda b, i: (b, i, 0)),
            pl.BlockSpec((1, N, D), lambda b, i: (b, 0, 0)),
            pl.BlockSpec((ROWS, D), lambda b, i: (i, 0)),
            pl.BlockSpec((N, D), lambda b, i: (0, 0)),
            pl.BlockSpec((ROWS, D), lambda b, i: (i, 0)),
            pl.BlockSpec((N, D), lambda b, i: (0, 0)),
        ],
        out_specs=pl.BlockSpec((1, ROWS, N), lambda b, i: (b, i, 0)),
        out_shape=jax.ShapeDtypeStruct((B, N, N), jnp.float32),
        compiler_params=pltpu.CompilerParams(
            dimension_semantics=("parallel", "parallel")),
    )(xn, xn, m1, m1, m2, m2)
    return out


# speedup vs baseline: 18.8112x; 1.0491x over previous
import functools

import jax
import jax.numpy as jnp
from jax.experimental import pallas as pl
from jax.experimental.pallas import tpu as pltpu

N = 2048
D = 64
K = 16
ALPHA = 0.2
ROWS = 256


def _prep_kernel(x_ref, e1_ref, e2_ref, t1_ref, t2_ref,
                 xn_ref, m1_ref, m2_ref):
    x = x_ref[0]
    acc = x[0]
    for t in range(1, x.shape[0]):
        acc = acc + x[t]
    xm = acc / x.shape[0]
    norm = jnp.sqrt(jnp.sum(xm * xm, axis=-1, keepdims=True))
    xn_ref[0] = xm / jnp.maximum(norm, 1e-8)
    m1_ref[...] = jnp.tanh(ALPHA * jnp.dot(e1_ref[...], t1_ref[...],
                                           preferred_element_type=jnp.float32))
    m2_ref[...] = jnp.tanh(ALPHA * jnp.dot(e2_ref[...], t2_ref[...],
                                           preferred_element_type=jnp.float32))


def _main_kernel(xnr_ref, xnf_ref, m1r_ref, m1f_ref, m2r_ref, m2f_ref,
                 out_ref, dyn_ref):
    @pl.when(pl.program_id(1) == 0)
    def _():
        a = jax.lax.dot_general(m1r_ref[...], m2f_ref[...],
                                (((1,), (1,)), ((), ())),
                                preferred_element_type=jnp.float32)
        bm = jax.lax.dot_general(m2r_ref[...], m1f_ref[...],
                                 (((1,), (1,)), ((), ())),
                                 preferred_element_type=jnp.float32)
        dyn_ref[...] = jax.nn.relu(jnp.tanh(ALPHA * (a - bm)))

    xr = xnr_ref[0]
    xf = xnf_ref[0]
    sim = jax.lax.dot_general(xr, xf, (((1,), (1,)), ((), ())),
                              preferred_element_type=jnp.float32)
    v = sim
    for _ in range(K - 1):
        m = jnp.max(v, axis=-1, keepdims=True)
        v = jnp.where(v >= m, -2.0, v)
    thresh = jnp.max(v, axis=-1, keepdims=True)
    out_ref[0] = jnp.where(sim >= thresh, dyn_ref[...] * sim, 0.0)


@jax.jit
def kernel(X, E1, E2, theta1, theta2):
    B, T = X.shape[0], X.shape[1]
    xn, m1, m2 = pl.pallas_call(
        _prep_kernel,
        grid=(B,),
        in_specs=[
            pl.BlockSpec((1, T, N, D), lambda b: (b, 0, 0, 0)),
            pl.BlockSpec((N, D), lambda b: (0, 0)),
            pl.BlockSpec((N, D), lambda b: (0, 0)),
            pl.BlockSpec((D, D), lambda b: (0, 0)),
            pl.BlockSpec((D, D), lambda b: (0, 0)),
        ],
        out_specs=[
            pl.BlockSpec((1, N, D), lambda b: (b, 0, 0)),
            pl.BlockSpec((N, D), lambda b: (0, 0)),
            pl.BlockSpec((N, D), lambda b: (0, 0)),
        ],
        out_shape=[
            jax.ShapeDtypeStruct((B, N, D), jnp.float32),
            jax.ShapeDtypeStruct((N, D), jnp.float32),
            jax.ShapeDtypeStruct((N, D), jnp.float32),
        ],
    )(X, E1, E2, theta1, theta2)

    out = pl.pallas_call(
        _main_kernel,
        grid=(N // ROWS, B),
        in_specs=[
            pl.BlockSpec((1, ROWS, D), lambda i, b: (b, i, 0)),
            pl.BlockSpec((1, N, D), lambda i, b: (b, 0, 0)),
            pl.BlockSpec((ROWS, D), lambda i, b: (i, 0)),
            pl.BlockSpec((N, D), lambda i, b: (0, 0)),
            pl.BlockSpec((ROWS, D), lambda i, b: (i, 0)),
            pl.BlockSpec((N, D), lambda i, b: (0, 0)),
        ],
        out_specs=pl.BlockSpec((1, ROWS, N), lambda i, b: (b, i, 0)),
        out_shape=jax.ShapeDtypeStruct((B, N, N), jnp.float32),
        scratch_shapes=[pltpu.VMEM((ROWS, N), jnp.float32)],
        compiler_params=pltpu.CompilerParams(
            dimension_semantics=("arbitrary", "arbitrary")),
    )(xn, xn, m1, m1, m2, m2)
    return out


# speedup vs baseline: 19.5238x; 1.0379x over previous
import functools

import jax
import jax.numpy as jnp
from jax.experimental import pallas as pl
from jax.experimental.pallas import tpu as pltpu

N = 2048
D = 64
K = 16
ALPHA = 0.2
ROWS = 256


def _prep_kernel(x_ref, e1_ref, e2_ref, t1_ref, t2_ref,
                 xn_ref, m1_ref, m2_ref):
    x = x_ref[0]
    acc = x[0]
    for t in range(1, x.shape[0]):
        acc = acc + x[t]
    xm = acc / x.shape[0]
    norm = jnp.sqrt(jnp.sum(xm * xm, axis=-1, keepdims=True))
    xn_ref[0] = xm / jnp.maximum(norm, 1e-8)
    m1_ref[...] = jnp.tanh(ALPHA * jnp.dot(e1_ref[...], t1_ref[...],
                                           preferred_element_type=jnp.float32))
    m2_ref[...] = jnp.tanh(ALPHA * jnp.dot(e2_ref[...], t2_ref[...],
                                           preferred_element_type=jnp.float32))


def _main_kernel(xnr_ref, xnf_ref, m1r_ref, m1f_ref, m2r_ref, m2f_ref,
                 out_ref, dyn_ref):
    @pl.when(pl.program_id(1) == 0)
    def _():
        a = jax.lax.dot_general(m1r_ref[...], m2f_ref[...],
                                (((1,), (1,)), ((), ())),
                                preferred_element_type=jnp.float32)
        bm = jax.lax.dot_general(m2r_ref[...], m1f_ref[...],
                                 (((1,), (1,)), ((), ())),
                                 preferred_element_type=jnp.float32)
        dyn_ref[...] = jax.nn.relu(jnp.tanh(ALPHA * (a - bm)))

    xr = xnr_ref[0]
    xf = xnf_ref[0]
    sim = jax.lax.dot_general(xr, xf, (((1,), (1,)), ((), ())),
                              preferred_element_type=jnp.float32)
    def _top2(v):
        a1 = v[:, 0:128]
        a2 = jnp.full_like(a1, -2.0)
        for s in range(1, 16):
            x = v[:, s * 128:(s + 1) * 128]
            lo = jnp.minimum(a1, x)
            a1 = jnp.maximum(a1, x)
            a2 = jnp.maximum(a2, lo)
        m1 = jnp.max(a1, axis=-1, keepdims=True)
        rest = jnp.where(a1 >= m1, -2.0, a1)
        s1 = jnp.max(rest, axis=-1, keepdims=True)
        s2 = jnp.max(a2, axis=-1, keepdims=True)
        return jnp.maximum(s1, s2)

    v = sim
    for _ in range(K // 2 - 1):
        m2 = _top2(v)
        v = jnp.where(v >= m2, -2.0, v)
    thresh = _top2(v)
    out_ref[0] = jnp.where(sim >= thresh, dyn_ref[...] * sim, 0.0)


@jax.jit
def kernel(X, E1, E2, theta1, theta2):
    B, T = X.shape[0], X.shape[1]
    xn, m1, m2 = pl.pallas_call(
        _prep_kernel,
        grid=(B,),
        in_specs=[
            pl.BlockSpec((1, T, N, D), lambda b: (b, 0, 0, 0)),
            pl.BlockSpec((N, D), lambda b: (0, 0)),
            pl.BlockSpec((N, D), lambda b: (0, 0)),
            pl.BlockSpec((D, D), lambda b: (0, 0)),
            pl.BlockSpec((D, D), lambda b: (0, 0)),
        ],
        out_specs=[
            pl.BlockSpec((1, N, D), lambda b: (b, 0, 0)),
            pl.BlockSpec((N, D), lambda b: (0, 0)),
            pl.BlockSpec((N, D), lambda b: (0, 0)),
        ],
        out_shape=[
            jax.ShapeDtypeStruct((B, N, D), jnp.float32),
            jax.ShapeDtypeStruct((N, D), jnp.float32),
            jax.ShapeDtypeStruct((N, D), jnp.float32),
        ],
    )(X, E1, E2, theta1, theta2)

    out = pl.pallas_call(
        _main_kernel,
        grid=(N // ROWS, B),
        in_specs=[
            pl.BlockSpec((1, ROWS, D), lambda i, b: (b, i, 0)),
            pl.BlockSpec((1, N, D), lambda i, b: (b, 0, 0)),
            pl.BlockSpec((ROWS, D), lambda i, b: (i, 0)),
            pl.BlockSpec((N, D), lambda i, b: (0, 0)),
            pl.BlockSpec((ROWS, D), lambda i, b: (i, 0)),
            pl.BlockSpec((N, D), lambda i, b: (0, 0)),
        ],
        out_specs=pl.BlockSpec((1, ROWS, N), lambda i, b: (b, i, 0)),
        out_shape=jax.ShapeDtypeStruct((B, N, N), jnp.float32),
        scratch_shapes=[pltpu.VMEM((ROWS, N), jnp.float32)],
        compiler_params=pltpu.CompilerParams(
            dimension_semantics=("arbitrary", "arbitrary")),
    )(xn, xn, m1, m1, m2, m2)
    return out


# speedup vs baseline: 21.9138x; 1.1224x over previous
import functools

import jax
import jax.numpy as jnp
from jax.experimental import pallas as pl
from jax.experimental.pallas import tpu as pltpu

N = 2048
D = 64
K = 16
ALPHA = 0.2
ROWS = 512


def _prep_kernel(x_ref, e1_ref, e2_ref, t1_ref, t2_ref,
                 xn_ref, m1_ref, m2_ref):
    x = x_ref[0]
    acc = x[0]
    for t in range(1, x.shape[0]):
        acc = acc + x[t]
    xm = acc / x.shape[0]
    norm = jnp.sqrt(jnp.sum(xm * xm, axis=-1, keepdims=True))
    xn_ref[0] = xm / jnp.maximum(norm, 1e-8)
    m1_ref[...] = jnp.tanh(ALPHA * jnp.dot(e1_ref[...], t1_ref[...],
                                           preferred_element_type=jnp.float32))
    m2_ref[...] = jnp.tanh(ALPHA * jnp.dot(e2_ref[...], t2_ref[...],
                                           preferred_element_type=jnp.float32))


def _main_kernel(xnr_ref, xnf_ref, m1r_ref, m1f_ref, m2r_ref, m2f_ref,
                 out_ref, dyn_ref):
    @pl.when(pl.program_id(1) == 0)
    def _():
        a = jax.lax.dot_general(m1r_ref[...], m2f_ref[...],
                                (((1,), (1,)), ((), ())),
                                preferred_element_type=jnp.float32)
        bm = jax.lax.dot_general(m2r_ref[...], m1f_ref[...],
                                 (((1,), (1,)), ((), ())),
                                 preferred_element_type=jnp.float32)
        dyn_ref[...] = jax.nn.relu(jnp.tanh(ALPHA * (a - bm)))

    xr = xnr_ref[0]
    xf = xnf_ref[0]
    sim = jax.lax.dot_general(xr, xf, (((1,), (1,)), ((), ())),
                              preferred_element_type=jnp.float32)
    def _top2(v):
        a1 = v[:, 0:128]
        a2 = jnp.full_like(a1, -2.0)
        for s in range(1, 16):
            x = v[:, s * 128:(s + 1) * 128]
            lo = jnp.minimum(a1, x)
            a1 = jnp.maximum(a1, x)
            a2 = jnp.maximum(a2, lo)
        m1 = jnp.max(a1, axis=-1, keepdims=True)
        rest = jnp.where(a1 >= m1, -2.0, a1)
        s1 = jnp.max(rest, axis=-1, keepdims=True)
        s2 = jnp.max(a2, axis=-1, keepdims=True)
        return jnp.maximum(s1, s2)

    v = sim
    for _ in range(K // 2 - 1):
        m2 = _top2(v)
        v = jnp.where(v >= m2, -2.0, v)
    thresh = _top2(v)
    out_ref[0] = jnp.where(sim >= thresh, dyn_ref[...] * sim, 0.0)


@jax.jit
def kernel(X, E1, E2, theta1, theta2):
    B, T = X.shape[0], X.shape[1]
    xn, m1, m2 = pl.pallas_call(
        _prep_kernel,
        grid=(B,),
        in_specs=[
            pl.BlockSpec((1, T, N, D), lambda b: (b, 0, 0, 0)),
            pl.BlockSpec((N, D), lambda b: (0, 0)),
            pl.BlockSpec((N, D), lambda b: (0, 0)),
            pl.BlockSpec((D, D), lambda b: (0, 0)),
            pl.BlockSpec((D, D), lambda b: (0, 0)),
        ],
        out_specs=[
            pl.BlockSpec((1, N, D), lambda b: (b, 0, 0)),
            pl.BlockSpec((N, D), lambda b: (0, 0)),
            pl.BlockSpec((N, D), lambda b: (0, 0)),
        ],
        out_shape=[
            jax.ShapeDtypeStruct((B, N, D), jnp.float32),
            jax.ShapeDtypeStruct((N, D), jnp.float32),
            jax.ShapeDtypeStruct((N, D), jnp.float32),
        ],
    )(X, E1, E2, theta1, theta2)

    out = pl.pallas_call(
        _main_kernel,
        grid=(N // ROWS, B),
        in_specs=[
            pl.BlockSpec((1, ROWS, D), lambda i, b: (b, i, 0)),
            pl.BlockSpec((1, N, D), lambda i, b: (b, 0, 0)),
            pl.BlockSpec((ROWS, D), lambda i, b: (i, 0)),
            pl.BlockSpec((N, D), lambda i, b: (0, 0)),
            pl.BlockSpec((ROWS, D), lambda i, b: (i, 0)),
            pl.BlockSpec((N, D), lambda i, b: (0, 0)),
        ],
        out_specs=pl.BlockSpec((1, ROWS, N), lambda i, b: (b, i, 0)),
        out_shape=jax.ShapeDtypeStruct((B, N, N), jnp.float32),
        scratch_shapes=[pltpu.VMEM((ROWS, N), jnp.float32)],
        compiler_params=pltpu.CompilerParams(
            dimension_semantics=("arbitrary", "arbitrary")),
    )(xn, xn, m1, m1, m2, m2)
    return out


# speedup vs baseline: 21.9427x; 1.0013x over previous
import jax
import jax.numpy as jnp
from jax.experimental import pallas as pl
from jax.experimental.pallas import tpu as pltpu

N = 2048
D = 64
K = 16
ALPHA = 0.2
ROWS = 512


def _prep_kernel(x_ref, e1_ref, e2_ref, t1_ref, t2_ref,
                 xn_ref, m1_ref, m2_ref):
    x = x_ref[0]
    acc = x[0]
    for t in range(1, x.shape[0]):
        acc = acc + x[t]
    xm = acc / x.shape[0]
    norm = jnp.sqrt(jnp.sum(xm * xm, axis=-1, keepdims=True))
    xn_ref[0] = xm / jnp.maximum(norm, 1e-8)
    m1_ref[...] = jnp.tanh(ALPHA * jnp.dot(e1_ref[...], t1_ref[...],
                                           preferred_element_type=jnp.float32))
    m2_ref[...] = jnp.tanh(ALPHA * jnp.dot(e2_ref[...], t2_ref[...],
                                           preferred_element_type=jnp.float32))


def _main_kernel(xnr_ref, xnf_ref, m1r_ref, m1f_ref, m2r_ref, m2f_ref,
                 out_ref, dyn_ref):
    @pl.when(pl.program_id(1) == 0)
    def _():
        a = jax.lax.dot_general(m1r_ref[...], m2f_ref[...],
                                (((1,), (1,)), ((), ())),
                                preferred_element_type=jnp.float32)
        bm = jax.lax.dot_general(m2r_ref[...], m1f_ref[...],
                                 (((1,), (1,)), ((), ())),
                                 preferred_element_type=jnp.float32)
        dyn_ref[...] = jax.nn.relu(jnp.tanh(ALPHA * (a - bm)))

    xr = xnr_ref[0]
    xf = xnf_ref[0]
    sim = jax.lax.dot_general(xr, xf, (((1,), (1,)), ((), ())),
                              preferred_element_type=jnp.float32)
    def _top2(v, thr):
        a1 = None
        a2 = None
        for s in range(16):
            x = v[:, s * 128:(s + 1) * 128]
            if thr is not None:
                x = jnp.where(x >= thr, -2.0, x)
            if a1 is None:
                a1, a2 = x, jnp.full_like(x, -2.0)
            else:
                lo = jnp.minimum(a1, x)
                a1 = jnp.maximum(a1, x)
                a2 = jnp.maximum(a2, lo)
        m1 = jnp.max(a1, axis=-1, keepdims=True)
        rest = jnp.where(a1 >= m1, -2.0, a1)
        s1 = jnp.max(rest, axis=-1, keepdims=True)
        s2 = jnp.max(a2, axis=-1, keepdims=True)
        return jnp.maximum(s1, s2)

    thr = None
    for _ in range(K // 2):
        thr = _top2(sim, thr)
    thresh = thr
    out_ref[0] = jnp.where(sim >= thresh, dyn_ref[...] * sim, 0.0)


@jax.jit
def kernel(X, E1, E2, theta1, theta2):
    B, T = X.shape[0], X.shape[1]
    xn, m1, m2 = pl.pallas_call(
        _prep_kernel,
        grid=(B,),
        in_specs=[
            pl.BlockSpec((1, T, N, D), lambda b: (b, 0, 0, 0)),
            pl.BlockSpec((N, D), lambda b: (0, 0)),
            pl.BlockSpec((N, D), lambda b: (0, 0)),
            pl.BlockSpec((D, D), lambda b: (0, 0)),
            pl.BlockSpec((D, D), lambda b: (0, 0)),
        ],
        out_specs=[
            pl.BlockSpec((1, N, D), lambda b: (b, 0, 0)),
            pl.BlockSpec((N, D), lambda b: (0, 0)),
            pl.BlockSpec((N, D), lambda b: (0, 0)),
        ],
        out_shape=[
            jax.ShapeDtypeStruct((B, N, D), jnp.float32),
            jax.ShapeDtypeStruct((N, D), jnp.float32),
            jax.ShapeDtypeStruct((N, D), jnp.float32),
        ],
    )(X, E1, E2, theta1, theta2)

    out = pl.pallas_call(
        _main_kernel,
        grid=(N // ROWS, B),
        in_specs=[
            pl.BlockSpec((1, ROWS, D), lambda i, b: (b, i, 0)),
            pl.BlockSpec((1, N, D), lambda i, b: (b, 0, 0)),
            pl.BlockSpec((ROWS, D), lambda i, b: (i, 0)),
            pl.BlockSpec((N, D), lambda i, b: (0, 0)),
            pl.BlockSpec((ROWS, D), lambda i, b: (i, 0)),
            pl.BlockSpec((N, D), lambda i, b: (0, 0)),
        ],
        out_specs=pl.BlockSpec((1, ROWS, N), lambda i, b: (b, i, 0)),
        out_shape=jax.ShapeDtypeStruct((B, N, N), jnp.float32),
        scratch_shapes=[pltpu.VMEM((ROWS, N), jnp.float32)],
        compiler_params=pltpu.CompilerParams(
            dimension_semantics=("arbitrary", "arbitrary")),
    )(xn, xn, m1, m1, m2, m2)
    return out
